# speedup vs baseline: 4.2418x; 4.2418x over previous
import jax
import jax.numpy as jnp
from jax.experimental import pallas as pl
from jax.experimental.pallas import tpu as pltpu

E = 64
K = 8
R = 16
TEMP = 0.5
TILE = 512


def _h_kernel(x_ref, wt64_ref, b64_ref, h_ref, mx_ref):
    i = pl.program_id(0)
    h = jnp.dot(x_ref[...], wt64_ref[...], preferred_element_type=jnp.float32)
    h = h + b64_ref[...]
    h_ref[...] = h
    tmax = jnp.max(jnp.abs(h))

    @pl.when(i == 0)
    def _():
        mx_ref[0, 0] = tmax

    @pl.when(i != 0)
    def _():
        mx_ref[0, 0] = jnp.maximum(mx_ref[0, 0], tmax)


def _main_kernel(mx_ref, x_ref, h_ref, wt_ref, b_ref, a_ref, bm_ref,
                 lime_ref, o_ref):
    x = x_ref[...]
    h = h_ref[...]
    scale = jnp.maximum(mx_ref[0, 0], 1e-6)
    inv = (1.0 / TEMP) / scale
    logits = h * inv
    lmax = jnp.max(logits, axis=-1, keepdims=True)
    ex = jnp.exp(logits - lmax)
    probs = ex / jnp.sum(ex, axis=-1, keepdims=True)

    iota = jax.lax.broadcasted_iota(jnp.int32, probs.shape, 1)
    cur = probs
    mask = jnp.zeros(probs.shape, jnp.bool_)
    for _ in range(K):
        mval = jnp.max(cur, axis=-1, keepdims=True)
        ism = cur == mval
        idx = jnp.min(jnp.where(ism, iota, E), axis=-1, keepdims=True)
        sel = iota == idx
        mask = jnp.logical_or(mask, sel)
        cur = jnp.where(sel, -jnp.inf, cur)

    w = jnp.where(mask, probs, 0.0)
    s = jnp.clip(jnp.sum(w, axis=-1, keepdims=True), 1e-9, None)
    wn = w / s
    p_mix = jnp.dot(wn, lime_ref[...], preferred_element_type=jnp.float32)

    base = jnp.dot(x, wt_ref[...], preferred_element_type=jnp.float32)
    base = base + b_ref[...]
    u = jnp.dot(x, a_ref[...], preferred_element_type=jnp.float32)
    delta = jnp.dot(u * p_mix, bm_ref[...],
                    preferred_element_type=jnp.float32)
    o_ref[...] = base + delta


def kernel(x, W, b, A, Bm, LiMEs):
    Bb, T, D_in = x.shape
    D_out = W.shape[0]
    N = Bb * T
    x2 = x.reshape(N, D_in)
    Wt = W.T

    h, mx = pl.pallas_call(
        _h_kernel,
        grid=(N // TILE,),
        in_specs=[
            pl.BlockSpec((TILE, D_in), lambda i: (i, 0)),
            pl.BlockSpec((D_in, E), lambda i: (0, 0)),
            pl.BlockSpec((1, E), lambda i: (0, 0)),
        ],
        out_specs=[
            pl.BlockSpec((TILE, E), lambda i: (i, 0)),
            pl.BlockSpec(memory_space=pltpu.SMEM),
        ],
        out_shape=[
            jax.ShapeDtypeStruct((N, E), jnp.float32),
            jax.ShapeDtypeStruct((1, 1), jnp.float32),
        ],
    )(x2, Wt[:, :E], b[:E].reshape(1, E))

    out = pl.pallas_call(
        _main_kernel,
        grid=(N // TILE,),
        in_specs=[
            pl.BlockSpec(memory_space=pltpu.SMEM),
            pl.BlockSpec((TILE, D_in), lambda i: (i, 0)),
            pl.BlockSpec((TILE, E), lambda i: (i, 0)),
            pl.BlockSpec((D_in, D_out), lambda i: (0, 0)),
            pl.BlockSpec((1, D_out), lambda i: (0, 0)),
            pl.BlockSpec((D_in, R), lambda i: (0, 0)),
            pl.BlockSpec((R, D_out), lambda i: (0, 0)),
            pl.BlockSpec((E, R), lambda i: (0, 0)),
        ],
        out_specs=pl.BlockSpec((TILE, D_out), lambda i: (i, 0)),
        out_shape=jax.ShapeDtypeStruct((N, D_out), jnp.float32),
    )(mx, x2, h, Wt, b.reshape(1, D_out), A, Bm, LiMEs)

    return out.reshape(Bb, T, D_out)


# speedup vs baseline: 5.7561x; 1.3570x over previous
import jax
import jax.numpy as jnp
from jax.experimental import pallas as pl
from jax.experimental.pallas import tpu as pltpu

E = 64
K = 8
R = 16
TEMP = 0.5
TILE = 512


def _h_kernel(x_ref, wt64_ref, b64_ref, h_ref, mx_ref):
    i = pl.program_id(0)
    h = jnp.dot(x_ref[...], wt64_ref[...], preferred_element_type=jnp.float32)
    h = h + b64_ref[...]
    h_ref[...] = h
    tmax = jnp.max(jnp.abs(h))

    @pl.when(i == 0)
    def _():
        mx_ref[0, 0] = tmax

    @pl.when(i != 0)
    def _():
        mx_ref[0, 0] = jnp.maximum(mx_ref[0, 0], tmax)


def _main_kernel(mx_ref, x_ref, h_ref, wt_ref, b_ref, a_ref, bm_ref,
                 lime_ref, o_ref):
    x = x_ref[...]
    h = h_ref[...]
    scale = jnp.maximum(mx_ref[0, 0], 1e-6)
    inv = (1.0 / TEMP) / scale
    ex = jnp.exp(h * inv)

    bits = jax.lax.bitcast_convert_type(ex, jnp.int32)
    iota = jax.lax.broadcasted_iota(jnp.int32, ex.shape, 1)
    cur = jax.lax.bitcast_convert_type((bits & -64) | (63 - iota),
                                       jnp.float32)
    for _ in range(K):
        mval = jnp.max(cur, axis=-1, keepdims=True)
        cur = jnp.where(cur == mval, 0.0, cur)

    w = jnp.where(cur == 0.0, ex, 0.0)
    s = jnp.sum(w, axis=-1, keepdims=True)
    wn = w / s
    p_mix = jnp.dot(wn, lime_ref[...], preferred_element_type=jnp.float32)

    base = jnp.dot(x, wt_ref[...], preferred_element_type=jnp.float32)
    base = base + b_ref[...]
    u = jnp.dot(x, a_ref[...], preferred_element_type=jnp.float32)
    delta = jnp.dot(u * p_mix, bm_ref[...],
                    preferred_element_type=jnp.float32)
    o_ref[...] = base + delta


def kernel(x, W, b, A, Bm, LiMEs):
    Bb, T, D_in = x.shape
    D_out = W.shape[0]
    N = Bb * T
    x2 = x.reshape(N, D_in)
    Wt = W.T

    h, mx = pl.pallas_call(
        _h_kernel,
        grid=(N // TILE,),
        in_specs=[
            pl.BlockSpec((TILE, D_in), lambda i: (i, 0)),
            pl.BlockSpec((D_in, E), lambda i: (0, 0)),
            pl.BlockSpec((1, E), lambda i: (0, 0)),
        ],
        out_specs=[
            pl.BlockSpec((TILE, E), lambda i: (i, 0)),
            pl.BlockSpec(memory_space=pltpu.SMEM),
        ],
        out_shape=[
            jax.ShapeDtypeStruct((N, E), jnp.float32),
            jax.ShapeDtypeStruct((1, 1), jnp.float32),
        ],
    )(x2, Wt[:, :E], b[:E].reshape(1, E))

    out = pl.pallas_call(
        _main_kernel,
        grid=(N // TILE,),
        in_specs=[
            pl.BlockSpec(memory_space=pltpu.SMEM),
            pl.BlockSpec((TILE, D_in), lambda i: (i, 0)),
            pl.BlockSpec((TILE, E), lambda i: (i, 0)),
            pl.BlockSpec((D_in, D_out), lambda i: (0, 0)),
            pl.BlockSpec((1, D_out), lambda i: (0, 0)),
            pl.BlockSpec((D_in, R), lambda i: (0, 0)),
            pl.BlockSpec((R, D_out), lambda i: (0, 0)),
            pl.BlockSpec((E, R), lambda i: (0, 0)),
        ],
        out_specs=pl.BlockSpec((TILE, D_out), lambda i: (i, 0)),
        out_shape=jax.ShapeDtypeStruct((N, D_out), jnp.float32),
    )(mx, x2, h, Wt, b.reshape(1, D_out), A, Bm, LiMEs)

    return out.reshape(Bb, T, D_out)
